# Initial kernel scaffold; baseline (speedup 1.0000x reference)
import jax
import jax.numpy as jnp
from jax.experimental import pallas as pl


def kernel(x_t, x_s, edge_index, edge_index_t, edge_weight_t, edge_index_s, edge_weight_s, params):
    raise NotImplementedError("write your pallas kernel here")


# speedup vs baseline: 1.0417x; 1.0417x over previous
import functools

import jax
import jax.numpy as jnp
from jax import lax
from jax.experimental import pallas as pl
from jax.experimental.pallas import tpu as pltpu

N_NODES = 10000
N_EDGES = 160000
LEAK = 0.1
BN_EPS = 1e-5


def _leaky(x):
    return jnp.where(x > 0, x, LEAK * x)


def _row_block(r):
    for cand in (4000, 2000, 1000, 500, 250, 125, 100, 50, 25, 10, 8):
        if r % cand == 0:
            return cand
    return r


def _fused_body(nx, subs, has_b, has_add, has_adddiv, has_outdiv, act,
                want_stats, *refs):
    i = 0
    xs = refs[i:i + nx]; i += nx
    srefs = []
    for k in range(nx):
        if subs[k]:
            srefs.append(refs[i]); i += 1
        else:
            srefs.append(None)
    ws = refs[i:i + nx]; i += nx
    b_ref = refs[i] if has_b else None; i += has_b
    add_ref = refs[i] if has_add else None; i += has_add
    adddiv_ref = refs[i] if has_adddiv else None; i += has_adddiv
    outdiv_ref = refs[i] if has_outdiv else None; i += has_outdiv
    y_ref = refs[i]; i += 1
    st_ref = refs[i] if want_stats else None

    acc = None
    for k in range(nx):
        v = xs[k][...]
        if srefs[k] is not None:
            v = v - srefs[k][...]
        t = jnp.dot(v, ws[k][...], preferred_element_type=jnp.float32)
        acc = t if acc is None else acc + t
    if has_b:
        acc = acc + b_ref[...]
    if has_add:
        a = add_ref[...]
        if has_adddiv:
            a = a / adddiv_ref[...]
        acc = acc + a
    if has_outdiv:
        acc = acc / outdiv_ref[...]
    if act == "relu":
        acc = jnp.maximum(acc, 0.0)
    elif act == "leaky":
        acc = _leaky(acc)
    y_ref[...] = acc
    if want_stats:
        s1 = jnp.sum(acc, axis=0)
        s2 = jnp.sum(acc * acc, axis=0)
        st_ref[...] = jnp.stack([s1, s2])[None]


def fused_rows(parts, b=None, add=None, add_rowdiv=None, out_rowdiv=None,
               act="none", want_stats=False):
    R = parts[0][0].shape[0]
    N = parts[0][2].shape[1]
    BR = _row_block(R)
    G = R // BR
    in_specs = []
    args = []
    for (x, _s, _w) in parts:
        in_specs.append(pl.BlockSpec((BR, x.shape[1]), lambda g: (g, 0)))
        args.append(x)
    for (x, s, _w) in parts:
        if s is not None:
            in_specs.append(pl.BlockSpec((BR, x.shape[1]), lambda g: (g, 0)))
            args.append(s)
    for (_x, _s, w) in parts:
        in_specs.append(pl.BlockSpec(w.shape, lambda g: (0, 0)))
        args.append(w)
    if b is not None:
        b2 = b.reshape(1, N)
        in_specs.append(pl.BlockSpec((1, N), lambda g: (0, 0)))
        args.append(b2)
    if add is not None:
        in_specs.append(pl.BlockSpec((BR, N), lambda g: (g, 0)))
        args.append(add)
    if add_rowdiv is not None:
        in_specs.append(pl.BlockSpec((BR, 1), lambda g: (g, 0)))
        args.append(add_rowdiv.reshape(R, 1))
    if out_rowdiv is not None:
        in_specs.append(pl.BlockSpec((BR, 1), lambda g: (g, 0)))
        args.append(out_rowdiv.reshape(R, 1))
    out_shape = [jax.ShapeDtypeStruct((R, N), jnp.float32)]
    out_specs = [pl.BlockSpec((BR, N), lambda g: (g, 0))]
    if want_stats:
        out_shape.append(jax.ShapeDtypeStruct((G, 2, N), jnp.float32))
        out_specs.append(pl.BlockSpec((1, 2, N), lambda g: (g, 0, 0)))
    body = functools.partial(
        _fused_body, len(parts), tuple(s is not None for (_x, s, _w) in parts),
        b is not None, add is not None, add_rowdiv is not None,
        out_rowdiv is not None, act, want_stats)
    res = pl.pallas_call(
        body,
        grid=(G,),
        in_specs=in_specs,
        out_specs=out_specs if want_stats else out_specs[0],
        out_shape=out_shape if want_stats else out_shape[0],
    )(*args)
    return res


def _bnfin_body(nrows, g_ref, bb_ref, st_ref, out_ref):
    s = jnp.sum(st_ref[...], axis=0)
    mean = s[0] / nrows
    var = s[1] / nrows - mean * mean
    scale = g_ref[...][0] / jnp.sqrt(var + BN_EPS)
    shift = bb_ref[...][0] - mean * scale
    out_ref[...] = jnp.stack([scale, shift])


def bn_finalize(stats, g, bb, nrows):
    G, _, N = stats.shape
    return pl.pallas_call(
        functools.partial(_bnfin_body, float(nrows)),
        in_specs=[pl.BlockSpec((1, N), lambda: (0, 0)),
                  pl.BlockSpec((1, N), lambda: (0, 0)),
                  pl.BlockSpec((G, 2, N), lambda: (0, 0, 0))],
        out_specs=pl.BlockSpec((2, N), lambda: (0, 0)),
        out_shape=jax.ShapeDtypeStruct((2, N), jnp.float32),
    )(g.reshape(1, N), bb.reshape(1, N), stats)


def _bnapply_body(y_ref, ss_ref, out_ref):
    ss = ss_ref[...]
    out_ref[...] = _leaky(y_ref[...] * ss[0] + ss[1])


def bn_apply_leaky(y, ss):
    R, N = y.shape
    BR = _row_block(R)
    return pl.pallas_call(
        _bnapply_body,
        grid=(R // BR,),
        in_specs=[pl.BlockSpec((BR, N), lambda g: (g, 0)),
                  pl.BlockSpec((2, N), lambda g: (0, 0))],
        out_specs=pl.BlockSpec((BR, N), lambda g: (g, 0)),
        out_shape=jax.ShapeDtypeStruct((R, N), jnp.float32),
    )(y, ss)


def _dot_body(r_ref, w_ref, out_ref):
    @pl.when(pl.program_id(0) == 0)
    def _init():
        out_ref[...] = jnp.zeros_like(out_ref)
    out_ref[...] += jnp.sum(r_ref[...] * w_ref[...]).reshape(1, 1)


def big_dot(r, w):
    R = r.shape[0]
    BR = _row_block(R)
    return pl.pallas_call(
        _dot_body,
        grid=(R // BR,),
        in_specs=[pl.BlockSpec((BR, 1), lambda g: (g, 0)),
                  pl.BlockSpec((BR, 1), lambda g: (g, 0))],
        out_specs=pl.BlockSpec((1, 1), lambda g: (0, 0)),
        out_shape=jax.ShapeDtypeStruct((1, 1), jnp.float32),
    )(r.reshape(R, 1), w.reshape(R, 1))


def spmv(ei, ew, x, n):
    return jax.ops.segment_sum(ew[:, None] * x[ei[1]], ei[0], num_segments=n)


def seg_sum_2idx(vals, idx0, idx1, n):
    return (jax.ops.segment_sum(vals, idx0, num_segments=n)
            + jax.ops.segment_sum(vals, idx1, num_segments=n))


def gather2(m, src, dst):
    return m[src] + m[dst]


def conv_block(x, ei, ew, p):
    s = spmv(ei, ew, x, x.shape[0])
    W0, W1 = p['W'][0], p['W'][1]
    y, st = fused_rows([(x, None, W0), (x, s, W1)], b=p['b'], want_stats=True)
    ss = bn_finalize(st, p['g'], p['bb'], x.shape[0])
    return bn_apply_leaky(y, ss)


def kernel(x_t, x_s, edge_index, edge_index_t, edge_weight_t,
           edge_index_s, edge_weight_s, params):
    src, dst = edge_index[0], edge_index[1]
    xt = x_t[:, 1:]
    xs = x_s[:, 1:]
    p = params
    xt = fused_rows([(xt, None, p['emb']['W'])], b=p['emb']['b'], act="relu")
    xt = conv_block(xt, edge_index_t, edge_weight_t, p['init_t'])
    xs = conv_block(xs, edge_index_s, edge_weight_s, p['init_s'])
    xt0, xs0 = xt, xs
    deg = seg_sum_2idx(jnp.ones((N_EDGES,), jnp.float32),
                       src, dst, N_NODES) + 1e-6
    for i in range(3):
        q = p['neint%d' % i]
        m_s = fused_rows([(xs0, None, q['Wst'])])
        nfe = seg_sum_2idx(m_s, src, dst, N_NODES)
        xt_n = fused_rows([(xt0, None, q['Wtt'])], add=nfe, add_rowdiv=deg,
                          act="leaky")
        m_t = fused_rows([(xt0, None, q['Wts'])], out_rowdiv=deg)
        xs_n = fused_rows([(xs0, None, q['Wss'])], add=gather2(m_t, src, dst),
                          act="leaky")
        xt = conv_block(xt_n, edge_index_t, edge_weight_t, p['nect%d' % i])
        xs = conv_block(xs_n, edge_index_s, edge_weight_s, p['necs%d' % i])
        xt0 = jnp.concatenate([xt0, xt], -1)
        xs0 = jnp.concatenate([xs0, xs], -1)
    rt = fused_rows([(xt, None, p['ro_t']['W'][0])], b=p['ro_t']['b'])
    rs = fused_rows([(xs, None, p['ro_s']['W'][0])], b=p['ro_s']['b'])
    wv = p['out']['W'][:, 0]
    tot = (big_dot(rs, wv[:N_EDGES]) + big_dot(rt, wv[N_EDGES:])
           + p['out']['b'])
    return tot.reshape(1, 1)
